# baseline (speedup 1.0000x reference)
import functools

import jax
import jax.numpy as jnp
from jax import lax
from jax.experimental import pallas as pl
from jax.experimental.pallas import tpu as pltpu
from jax.experimental.pallas import tpu_sc as plsc

_N = 4096
_D = 32
_NEG = 5
_NC = 2
_NS = 16
_NW = _NC * _NS
_POS_W = _N // _NW
_NEG_CH = 128
_NEG_W = _N * _NEG // _NW
_NEG_NCH = _NEG_W // _NEG_CH


def _sc_gather(u_emb, v_emb, pos_u, pos_v, neg_v3):
    mesh = plsc.VectorSubcoreMesh(core_axis_name="c", subcore_axis_name="s")

    @functools.partial(
        pl.kernel,
        mesh=mesh,
        compiler_params=pltpu.CompilerParams(use_tc_tiling_on_sc=False),
        out_type=[
            jax.ShapeDtypeStruct((_N, _D), jnp.float32),
            jax.ShapeDtypeStruct((_N, _D), jnp.float32),
            jax.ShapeDtypeStruct((_NW, _NEG_W, _D), jnp.float32),
        ],
        scratch_types=[
            pltpu.VMEM((_POS_W,), jnp.int32),
            pltpu.VMEM((_POS_W,), jnp.int32),
            pltpu.VMEM((_NEG_NCH, _NEG_CH), jnp.int32),
            pltpu.VMEM((_POS_W, _D), jnp.float32),
            pltpu.VMEM((_POS_W, _D), jnp.float32),
            pltpu.VMEM((_NEG_W, _D), jnp.float32),
            pltpu.SemaphoreType.DMA,
        ],
    )
    def k(u_hbm, v_hbm, pu_hbm, pv_hbm, nv_hbm, out_u, out_v, out_n,
          idx_u, idx_v, idx_n, rows_u, rows_v, rows_n, sem):
        wid = lax.axis_index("s") * _NC + lax.axis_index("c")
        base = wid * _POS_W
        pltpu.sync_copy(pu_hbm.at[pl.ds(base, _POS_W)], idx_u)
        pltpu.sync_copy(pv_hbm.at[pl.ds(base, _POS_W)], idx_v)
        pltpu.sync_copy(nv_hbm.at[wid], idx_n)
        copies = [
            pltpu.async_copy(u_hbm.at[idx_u], rows_u, sem),
            pltpu.async_copy(v_hbm.at[idx_v], rows_v, sem),
        ]
        for j in range(_NEG_NCH):
            copies.append(pltpu.async_copy(
                v_hbm.at[idx_n.at[j]],
                rows_n.at[pl.ds(j * _NEG_CH, _NEG_CH)],
                sem))
        for cp in copies:
            cp.wait()
        pltpu.sync_copy(rows_u, out_u.at[pl.ds(base, _POS_W)])
        pltpu.sync_copy(rows_v, out_v.at[pl.ds(base, _POS_W)])
        pltpu.sync_copy(rows_n, out_n.at[wid])

    return k(u_emb, v_emb, pos_u, pos_v, neg_v3)


def _log_sigmoid(x):
    return jnp.minimum(x, 0.0) - jnp.log(1.0 + jnp.exp(-jnp.abs(x)))


def _tc_loss_body(eu_ref, ev_ref, n0_ref, n1_ref, n2_ref, n3_ref, n4_ref,
                  out_ref):
    eu = eu_ref[...]
    score = jnp.sum(eu * ev_ref[...], axis=1)
    acc = _log_sigmoid(score)
    for nref in (n0_ref, n1_ref, n2_ref, n3_ref, n4_ref):
        acc = acc + _log_sigmoid(-jnp.sum(nref[...] * eu, axis=1))
    out_ref[0, 0] = jnp.sum(acc)


def _tc_loss(eu, ev, negs):
    return pl.pallas_call(
        _tc_loss_body,
        out_shape=jax.ShapeDtypeStruct((1, 1), jnp.float32),
        out_specs=pl.BlockSpec(memory_space=pltpu.SMEM),
    )(eu, ev, *negs)


def kernel(pos_u, pos_v, neg_v, batch_size, u_emb, v_emb):
    pu = pos_u.reshape(_N)
    pv = pos_v.reshape(_N)
    nv3 = neg_v.reshape(_NW, _NEG_NCH, _NEG_CH)
    eu, ev, en = _sc_gather(u_emb, v_emb, pu, pv, nv3)
    en = en.reshape(_N, _NEG, _D)
    negs = [en[:, k, :] for k in range(_NEG)]
    total = _tc_loss(eu, ev, negs)[0, 0]
    return -total / batch_size


# speedup vs baseline: 1.0351x; 1.0351x over previous
import functools

import jax
import jax.numpy as jnp
from jax import lax
from jax.experimental import pallas as pl
from jax.experimental.pallas import tpu as pltpu
from jax.experimental.pallas import tpu_sc as plsc

_N = 4096
_D = 32
_NEG = 5
_NC = 2
_NS = 16
_NW = _NC * _NS
_L = 16
_POS_W = _N // _NW
_NEG_CH = 128
_NEG_W = _N * _NEG // _NW
_NEG_NCH = _NEG_W // _NEG_CH


def _sc_scores(u_emb, v_emb, pos_u, pos_v, neg_v3):
    mesh = plsc.VectorSubcoreMesh(core_axis_name="c", subcore_axis_name="s")

    @functools.partial(
        pl.kernel,
        mesh=mesh,
        compiler_params=pltpu.CompilerParams(use_tc_tiling_on_sc=False,
                                             needs_layout_passes=False),
        out_type=[
            jax.ShapeDtypeStruct((_N,), jnp.float32),
            jax.ShapeDtypeStruct((_N * _NEG,), jnp.float32),
        ],
        scratch_types=[
            pltpu.VMEM((_POS_W,), jnp.int32),
            pltpu.VMEM((_POS_W,), jnp.int32),
            pltpu.VMEM((_NEG_NCH, _NEG_CH), jnp.int32),
            pltpu.VMEM((_POS_W, _D), jnp.float32),
            pltpu.VMEM((_POS_W, _D), jnp.float32),
            pltpu.VMEM((_NEG_W, _D), jnp.float32),
            pltpu.VMEM((_POS_W,), jnp.float32),
            pltpu.VMEM((_NEG_W,), jnp.float32),
            pltpu.SemaphoreType.DMA,
            pltpu.SemaphoreType.DMA,
        ],
    )
    def k(u_hbm, v_hbm, pu_hbm, pv_hbm, nv_hbm, out_s, out_ns,
          idx_u, idx_v, idx_n, rows_u, rows_v, rows_n, s_v, ns_v,
          sem_pos, sem_neg):
        wid = lax.axis_index("s") * _NC + lax.axis_index("c")
        base = wid * _POS_W
        pltpu.sync_copy(pu_hbm.at[pl.ds(base, _POS_W)], idx_u)
        pltpu.sync_copy(pv_hbm.at[pl.ds(base, _POS_W)], idx_v)
        pltpu.sync_copy(nv_hbm.at[wid], idx_n)
        cp_u = pltpu.async_copy(u_hbm.at[idx_u], rows_u, sem_pos)
        cp_v = pltpu.async_copy(v_hbm.at[idx_v], rows_v, sem_pos)
        cp_n = [
            pltpu.async_copy(v_hbm.at[idx_n.at[j]],
                             rows_n.at[pl.ds(j * _NEG_CH, _NEG_CH)],
                             sem_neg)
            for j in range(_NEG_NCH)
        ]
        cp_u.wait()
        cp_v.wait()

        lanes = lax.iota(jnp.int32, _L)

        def pos_group(g, _):
            rows = g * _L + lanes
            acc = jnp.zeros((_L,), jnp.float32)
            for d in range(_D):
                cols = jnp.full((_L,), d, jnp.int32)
                acc = acc + (plsc.load_gather(rows_u, [rows, cols])
                             * plsc.load_gather(rows_v, [rows, cols]))
            s_v[pl.ds(g * _L, _L)] = acc
            return _

        lax.fori_loop(0, _POS_W // _L, pos_group, 0, unroll=False)

        for cp in cp_n:
            cp.wait()

        def neg_group(g, _):
            jrows = g * _L + lanes
            urows = jrows // _NEG
            acc = jnp.zeros((_L,), jnp.float32)
            for d in range(_D):
                cols = jnp.full((_L,), d, jnp.int32)
                acc = acc + (plsc.load_gather(rows_n, [jrows, cols])
                             * plsc.load_gather(rows_u, [urows, cols]))
            ns_v[pl.ds(g * _L, _L)] = acc
            return _

        lax.fori_loop(0, _NEG_W // _L, neg_group, 0, unroll=False)

        pltpu.sync_copy(s_v, out_s.at[pl.ds(base, _POS_W)])
        pltpu.sync_copy(ns_v, out_ns.at[pl.ds(wid * _NEG_W, _NEG_W)])

    return k(u_emb, v_emb, pos_u, pos_v, neg_v3)


def _log_sigmoid(x):
    return jnp.minimum(x, 0.0) - jnp.log(1.0 + jnp.exp(-jnp.abs(x)))


def _tc_loss_body(s_ref, ns_ref, out_ref):
    total = jnp.sum(_log_sigmoid(s_ref[...]))
    total += jnp.sum(_log_sigmoid(-ns_ref[...]))
    out_ref[0, 0] = total


def _tc_loss(score2d, nscore2d):
    return pl.pallas_call(
        _tc_loss_body,
        out_shape=jax.ShapeDtypeStruct((1, 1), jnp.float32),
        out_specs=pl.BlockSpec(memory_space=pltpu.SMEM),
    )(score2d, nscore2d)


def kernel(pos_u, pos_v, neg_v, batch_size, u_emb, v_emb):
    pu = pos_u.reshape(_N)
    pv = pos_v.reshape(_N)
    nv3 = neg_v.reshape(_NW, _NEG_NCH, _NEG_CH)
    score, nscore = _sc_scores(u_emb, v_emb, pu, pv, nv3)
    total = _tc_loss(score.reshape(_N // 128, 128),
                     nscore.reshape(_N * _NEG // 128, 128))[0, 0]
    return -total / batch_size
